# baseline (device time: 538477 ns/iter reference)
import functools

import jax
import jax.numpy as jnp
from jax import lax
from jax.experimental import pallas as pl
from jax.experimental.pallas import tpu as pltpu

N_DEV = 16
B_LOC = 2
SQ = 256
SKV = 256
D_MODEL = 512
HQ_LOC = 4
DH = 64
D_HEADS_LOC = HQ_LOC * DH
BLK = 64
R_HOPS = 8
L_HOPS = 7
SUB = 4


def _body(x_ref, wq_ref, wo_ref, k_hbm, v_hbm, out_ref,
          wq_g, wo_g, k_sc, v_sc, kv_sems,
          rq_send, rq_recv, ro_send, ro_recv,
          lq_send, lq_recv, lo_send, lo_recv):
    my = lax.axis_index("i")
    left = lax.rem(my + N_DEV - 1, N_DEV)
    right = lax.rem(my + 1, N_DEV)

    kv_copies = {}

    def stage(slot):
        o = lax.rem(my + slot, N_DEV)
        cps = []
        for b in range(B_LOC):
            gb = my * B_LOC + b
            for j in range(HQ_LOC):
                head = HQ_LOC * o + j
                for hbm, sc in ((k_hbm, k_sc), (v_hbm, v_sc)):
                    c = pltpu.make_async_copy(
                        hbm.at[gb, :, head, :], sc.at[head, b],
                        kv_sems.at[slot])
                    c.start()
                    cps.append(c)
        kv_copies[slot] = cps

    stage(0)

    barrier_sem = pltpu.get_barrier_semaphore()
    for nbr in (left, right):
        pl.semaphore_signal(
            barrier_sem, inc=1,
            device_id=(nbr,), device_id_type=pl.DeviceIdType.MESH,
        )
    pl.semaphore_wait(barrier_sem, 2)

    wq_g[0] = wq_ref[:, :]
    wo_g[0] = wo_ref[:, :]
    stage(N_DEV - 1)
    stage(1)

    mask = (lax.broadcasted_iota(jnp.int32, (SQ, SKV), 1) // BLK) <= (
        lax.broadcasted_iota(jnp.int32, (SQ, SKV), 0) // BLK)

    x_all = x_ref[:, :, :].reshape(B_LOC * SQ, D_MODEL)
    acc = [jnp.zeros((B_LOC * SQ, D_MODEL), jnp.float32)]

    def contrib(slot):
        o = lax.rem(my + slot, N_DEV)
        wq_blk = wq_g[slot]
        wo_blk = wo_g[slot]
        for c in kv_copies.pop(slot):
            c.wait()
        q_all = jnp.dot(x_all, wq_blk,
                        preferred_element_type=jnp.float32)
        ctx_rows = []
        for b in range(B_LOC):
            ctx_parts = []
            for j in range(HQ_LOC):
                qj = q_all[SQ * b:SQ * (b + 1), DH * j:DH * (j + 1)]
                kj = k_sc[pl.ds(HQ_LOC * o + j, 1), b].reshape(
                    SKV, DH).astype(jnp.bfloat16)
                vj = v_sc[pl.ds(HQ_LOC * o + j, 1), b].reshape(
                    SKV, DH).astype(jnp.bfloat16)
                s_ = lax.dot_general(
                    qj.astype(jnp.bfloat16), kj, (((1,), (1,)), ((), ())),
                    preferred_element_type=jnp.float32) * 0.125
                w = jnp.exp(jnp.where(mask, s_, -1e9))
                wsum = jnp.sum(w, axis=1, keepdims=True)
                pv = jnp.dot(w.astype(jnp.bfloat16), vj,
                             preferred_element_type=jnp.float32)
                ctx_parts.append(pv / wsum)
            ctx_rows.append(jnp.concatenate(ctx_parts, axis=1))
        ctx_all = jnp.concatenate(ctx_rows, axis=0)
        acc[0] = acc[0] + jnp.dot(
            ctx_all.astype(jnp.bfloat16), wo_blk,
            preferred_element_type=jnp.float32)

    qrows = D_MODEL // SUB
    orows = D_HEADS_LOC // SUB

    def make_hop(h, rightward):
        if rightward:
            src, dst, tgt = (0 if h == 0 else N_DEV - h), N_DEV - 1 - h, right
            sems = (rq_send, rq_recv, ro_send, ro_recv)
        else:
            src, dst, tgt = (0 if h == 0 else h), h + 1, left
            sems = (lq_send, lq_recv, lo_send, lo_recv)
        subs = []
        for s in range(SUB):
            rq = pltpu.make_async_remote_copy(
                src_ref=wq_g.at[src, pl.ds(s * qrows, qrows)],
                dst_ref=wq_g.at[dst, pl.ds(s * qrows, qrows)],
                send_sem=sems[0].at[h, s], recv_sem=sems[1].at[h, s],
                device_id=(tgt,), device_id_type=pl.DeviceIdType.MESH,
            )
            ro = pltpu.make_async_remote_copy(
                src_ref=wo_g.at[src, pl.ds(s * orows, orows)],
                dst_ref=wo_g.at[dst, pl.ds(s * orows, orows)],
                send_sem=sems[2].at[h, s], recv_sem=sems[3].at[h, s],
                device_id=(tgt,), device_id_type=pl.DeviceIdType.MESH,
            )
            subs.append((rq, ro))
        return subs

    hops_r = [make_hop(h, True) for h in range(R_HOPS)]
    hops_l = [make_hop(h, False) for h in range(L_HOPS)]

    for rq, ro in hops_r[0] + hops_l[0]:
        rq.start()
        ro.start()
    contrib(0)

    for h in range(R_HOPS):
        if h <= R_HOPS - 2:
            stage(N_DEV - 2 - h)
        if h <= L_HOPS - 2:
            stage(h + 2)
        for s in range(SUB):
            rq, ro = hops_r[h][s]
            rq.wait_recv()
            ro.wait_recv()
            if h + 1 < R_HOPS:
                nq, no = hops_r[h + 1][s]
                nq.start()
                no.start()
            if h < L_HOPS:
                lq_, lo_ = hops_l[h][s]
                lq_.wait_recv()
                lo_.wait_recv()
                if h + 1 < L_HOPS:
                    nq, no = hops_l[h + 1][s]
                    nq.start()
                    no.start()
        for rq, ro in hops_r[h]:
            rq.wait_send()
            ro.wait_send()
        if h < L_HOPS:
            for rq, ro in hops_l[h]:
                rq.wait_send()
                ro.wait_send()
        contrib(N_DEV - 1 - h)
        if h < L_HOPS:
            contrib(h + 1)

    out_ref[:, :, :] = acc[0].reshape(B_LOC, SQ, D_MODEL)

    @functools.partial(pl.run_scoped, second_barrier=pltpu.SemaphoreType.REGULAR)
    def _(second_barrier):
        for nbr in (left, right):
            pl.semaphore_signal(
                second_barrier, inc=1,
                device_id=(nbr,), device_id_type=pl.DeviceIdType.MESH,
            )
        pl.semaphore_wait(second_barrier, 2)


def kernel(x, Wq, K_ext, V_ext, Wo):
    return pl.pallas_call(
        _body,
        out_shape=jax.ShapeDtypeStruct((B_LOC, SQ, D_MODEL), jnp.float32),
        in_specs=[pl.BlockSpec(memory_space=pltpu.VMEM)] * 3
        + [pl.BlockSpec(memory_space=pltpu.MemorySpace.HBM)] * 2,
        out_specs=pl.BlockSpec(memory_space=pltpu.VMEM),
        scratch_shapes=[
            pltpu.VMEM((N_DEV, D_MODEL, D_HEADS_LOC), jnp.bfloat16),
            pltpu.VMEM((N_DEV, D_HEADS_LOC, D_MODEL), jnp.bfloat16),
            pltpu.VMEM((N_DEV * HQ_LOC, B_LOC, SKV, DH), jnp.float32),
            pltpu.VMEM((N_DEV * HQ_LOC, B_LOC, SKV, DH), jnp.float32),
            pltpu.SemaphoreType.DMA((N_DEV,)),
            pltpu.SemaphoreType.DMA((R_HOPS, SUB)),
            pltpu.SemaphoreType.DMA((R_HOPS, SUB)),
            pltpu.SemaphoreType.DMA((R_HOPS, SUB)),
            pltpu.SemaphoreType.DMA((R_HOPS, SUB)),
            pltpu.SemaphoreType.DMA((L_HOPS, SUB)),
            pltpu.SemaphoreType.DMA((L_HOPS, SUB)),
            pltpu.SemaphoreType.DMA((L_HOPS, SUB)),
            pltpu.SemaphoreType.DMA((L_HOPS, SUB)),
        ],
        compiler_params=pltpu.CompilerParams(
            collective_id=0, vmem_limit_bytes=100 * 1024 * 1024
        ),
    )(x.astype(jnp.bfloat16), Wq.astype(jnp.bfloat16),
      Wo.astype(jnp.bfloat16), K_ext, V_ext)


# device time: 316463 ns/iter; 1.7015x vs baseline; 1.7015x over previous
import functools

import jax
import jax.numpy as jnp
from jax import lax
from jax.experimental import pallas as pl
from jax.experimental.pallas import tpu as pltpu

N_DEV = 16
B_LOC = 2
SQ = 256
SKV = 256
D_MODEL = 512
HQ_LOC = 4
DH = 64
D_HEADS_LOC = HQ_LOC * DH
BLK = 64
R_HOPS = 8
L_HOPS = 7
SUB = 4


def _body(x_ref, wq_ref, wo_ref, k_hbm, v_hbm, out_ref,
          wq_g, wo_g, k_raw, v_raw, k_sc, v_sc, kv_sem,
          rq_send, rq_recv, ro_send, ro_recv,
          lq_send, lq_recv, lo_send, lo_recv):
    my = lax.axis_index("i")
    left = lax.rem(my + N_DEV - 1, N_DEV)
    right = lax.rem(my + 1, N_DEV)

    kv_dmas = []
    for b in range(B_LOC):
        gb = my * B_LOC + b
        for t, (hbm, raw) in enumerate(((k_hbm, k_raw), (v_hbm, v_raw))):
            c = pltpu.make_async_copy(hbm.at[gb], raw.at[b],
                                      kv_sem.at[2 * b + t])
            c.start()
            kv_dmas.append(c)

    barrier_sem = pltpu.get_barrier_semaphore()
    for nbr in (left, right):
        pl.semaphore_signal(
            barrier_sem, inc=1,
            device_id=(nbr,), device_id_type=pl.DeviceIdType.MESH,
        )
    pl.semaphore_wait(barrier_sem, 2)

    wq_g[0] = wq_ref[:, :]
    wo_g[0] = wo_ref[:, :]

    mask = (lax.broadcasted_iota(jnp.int32, (SQ, SKV), 1) // BLK) <= (
        lax.broadcasted_iota(jnp.int32, (SQ, SKV), 0) // BLK)

    x_all = x_ref[:, :, :].reshape(B_LOC * SQ, D_MODEL)
    acc = [jnp.zeros((B_LOC * SQ, D_MODEL), jnp.float32)]

    def contrib(slot):
        o = lax.rem(my + slot, N_DEV)
        wq_blk = wq_g[slot]
        wo_blk = wo_g[slot]
        q_all = jnp.dot(x_all, wq_blk,
                        preferred_element_type=jnp.float32)
        ctx_rows = []
        for b in range(B_LOC):
            ctx_parts = []
            for j in range(HQ_LOC):
                qj = q_all[SQ * b:SQ * (b + 1), DH * j:DH * (j + 1)]
                kj = k_sc[pl.ds(HQ_LOC * o + j, 1), b].reshape(SKV, DH)
                vj = v_sc[pl.ds(HQ_LOC * o + j, 1), b].reshape(SKV, DH)
                s_ = lax.dot_general(
                    qj.astype(jnp.bfloat16), kj, (((1,), (1,)), ((), ())),
                    preferred_element_type=jnp.float32) * 0.125
                w = jnp.exp(jnp.where(mask, s_, -1e9))
                wsum = jnp.sum(w, axis=1, keepdims=True)
                pv = jnp.dot(w.astype(jnp.bfloat16), vj,
                             preferred_element_type=jnp.float32)
                ctx_parts.append(pv / wsum)
            ctx_rows.append(jnp.concatenate(ctx_parts, axis=1))
        ctx_all = jnp.concatenate(ctx_rows, axis=0)
        acc[0] = acc[0] + jnp.dot(
            ctx_all.astype(jnp.bfloat16), wo_blk,
            preferred_element_type=jnp.float32)

    qrows = D_MODEL // SUB
    orows = D_HEADS_LOC // SUB

    def make_hop(h, rightward):
        if rightward:
            src, dst, tgt = (0 if h == 0 else N_DEV - h), N_DEV - 1 - h, right
            sems = (rq_send, rq_recv, ro_send, ro_recv)
        else:
            src, dst, tgt = (0 if h == 0 else h), h + 1, left
            sems = (lq_send, lq_recv, lo_send, lo_recv)
        subs = []
        for s in range(SUB):
            rq = pltpu.make_async_remote_copy(
                src_ref=wq_g.at[src, pl.ds(s * qrows, qrows)],
                dst_ref=wq_g.at[dst, pl.ds(s * qrows, qrows)],
                send_sem=sems[0].at[h, s], recv_sem=sems[1].at[h, s],
                device_id=(tgt,), device_id_type=pl.DeviceIdType.MESH,
            )
            ro = pltpu.make_async_remote_copy(
                src_ref=wo_g.at[src, pl.ds(s * orows, orows)],
                dst_ref=wo_g.at[dst, pl.ds(s * orows, orows)],
                send_sem=sems[2].at[h, s], recv_sem=sems[3].at[h, s],
                device_id=(tgt,), device_id_type=pl.DeviceIdType.MESH,
            )
            subs.append((rq, ro))
        return subs

    hops_r = [make_hop(h, True) for h in range(R_HOPS)]
    hops_l = [make_hop(h, False) for h in range(L_HOPS)]

    for rq, ro in hops_r[0] + hops_l[0]:
        rq.start()
        ro.start()

    for c in kv_dmas:
        c.wait()
    for hh in range(N_DEV * HQ_LOC):
        for b in range(B_LOC):
            k_sc[hh, b] = k_raw[b, :, DH * hh:DH * (hh + 1)].astype(
                jnp.bfloat16)
            v_sc[hh, b] = v_raw[b, :, DH * hh:DH * (hh + 1)].astype(
                jnp.bfloat16)

    contrib(0)

    for h in range(R_HOPS):
        for s in range(SUB):
            rq, ro = hops_r[h][s]
            rq.wait_recv()
            ro.wait_recv()
            if h + 1 < R_HOPS:
                nq, no = hops_r[h + 1][s]
                nq.start()
                no.start()
            if h < L_HOPS:
                lq_, lo_ = hops_l[h][s]
                lq_.wait_recv()
                lo_.wait_recv()
                if h + 1 < L_HOPS:
                    nq, no = hops_l[h + 1][s]
                    nq.start()
                    no.start()
        for rq, ro in hops_r[h]:
            rq.wait_send()
            ro.wait_send()
        if h < L_HOPS:
            for rq, ro in hops_l[h]:
                rq.wait_send()
                ro.wait_send()
        contrib(N_DEV - 1 - h)
        if h < L_HOPS:
            contrib(h + 1)

    out_ref[:, :, :] = acc[0].reshape(B_LOC, SQ, D_MODEL)

    @functools.partial(pl.run_scoped, second_barrier=pltpu.SemaphoreType.REGULAR)
    def _(second_barrier):
        for nbr in (left, right):
            pl.semaphore_signal(
                second_barrier, inc=1,
                device_id=(nbr,), device_id_type=pl.DeviceIdType.MESH,
            )
        pl.semaphore_wait(second_barrier, 2)


def kernel(x, Wq, K_ext, V_ext, Wo):
    k2 = K_ext.reshape(32, SKV, N_DEV * HQ_LOC * DH)
    v2 = V_ext.reshape(32, SKV, N_DEV * HQ_LOC * DH)

    return pl.pallas_call(
        _body,
        out_shape=jax.ShapeDtypeStruct((B_LOC, SQ, D_MODEL), jnp.float32),
        in_specs=[pl.BlockSpec(memory_space=pltpu.VMEM)] * 3
        + [pl.BlockSpec(memory_space=pltpu.MemorySpace.HBM)] * 2,
        out_specs=pl.BlockSpec(memory_space=pltpu.VMEM),
        scratch_shapes=[
            pltpu.VMEM((N_DEV, D_MODEL, D_HEADS_LOC), jnp.bfloat16),
            pltpu.VMEM((N_DEV, D_HEADS_LOC, D_MODEL), jnp.bfloat16),
            pltpu.VMEM((B_LOC, SKV, N_DEV * HQ_LOC * DH), jnp.float32),
            pltpu.VMEM((B_LOC, SKV, N_DEV * HQ_LOC * DH), jnp.float32),
            pltpu.VMEM((N_DEV * HQ_LOC, B_LOC, SKV, DH), jnp.bfloat16),
            pltpu.VMEM((N_DEV * HQ_LOC, B_LOC, SKV, DH), jnp.bfloat16),
            pltpu.SemaphoreType.DMA((2 * B_LOC,)),
            pltpu.SemaphoreType.DMA((R_HOPS, SUB)),
            pltpu.SemaphoreType.DMA((R_HOPS, SUB)),
            pltpu.SemaphoreType.DMA((R_HOPS, SUB)),
            pltpu.SemaphoreType.DMA((R_HOPS, SUB)),
            pltpu.SemaphoreType.DMA((L_HOPS, SUB)),
            pltpu.SemaphoreType.DMA((L_HOPS, SUB)),
            pltpu.SemaphoreType.DMA((L_HOPS, SUB)),
            pltpu.SemaphoreType.DMA((L_HOPS, SUB)),
        ],
        compiler_params=pltpu.CompilerParams(
            collective_id=0, vmem_limit_bytes=100 * 1024 * 1024
        ),
    )(x.astype(jnp.bfloat16), Wq.astype(jnp.bfloat16),
      Wo.astype(jnp.bfloat16), k2, v2)
